# Initial kernel scaffold; baseline (speedup 1.0000x reference)
import jax
import jax.numpy as jnp
from jax.experimental import pallas as pl


def kernel(x, edge_index, batch, params):
    raise NotImplementedError("write your pallas kernel here")


# speedup vs baseline: 4.5055x; 4.5055x over previous
import functools

import jax
import jax.numpy as jnp
from jax import lax
from jax.experimental import pallas as pl
from jax.experimental.pallas import tpu as pltpu
from jax.experimental.pallas import tpu_sc as plsc

KW = 64
BN = 1000
H_DIM = 256
SC_CHUNK = 128
NSUB = 16
NCORE = 2


def _pad_up(v, m):
    return -(-v // m) * m


def _pad2(a, rows, cols):
    return jnp.pad(a, ((0, rows - a.shape[0]), (0, cols - a.shape[1])))


def _pad1(a, length):
    return jnp.pad(a, (0, length - a.shape[0]))


def _dot(a, b):
    return lax.dot(a, b, precision=lax.Precision.DEFAULT,
                   preferred_element_type=jnp.float32)


def _expm1(x):
    u = jnp.exp(x)
    um1 = u - 1.0
    return jnp.where(u == 1.0, x,
                     jnp.where(um1 == -1.0, -1.0, um1 * x / jnp.log(u)))


def _tree_conv(x, w_ref, s_out_p):
    xb = x.astype(jnp.bfloat16).astype(jnp.float32)

    def prod(k):
        wk = w_ref[0, k].astype(jnp.bfloat16).astype(jnp.float32)
        return xb[:, k:k + s_out_p] * wk

    stack = []
    for k in range(0, KW, 2):
        v = prod(k) + prod(k + 1)
        lvl = 1
        while stack and stack[-1][0] == lvl:
            v = stack.pop()[1] + v
            lvl += 1
        stack.append((lvl, v))
    v = stack.pop()[1]
    while stack:
        v = stack.pop()[1] + v
    return v


def _pre_call(u_ext, skip, wc, ln_g, ln_b, w_in, b_in, w_msg, *, s_real,
              interpret=False):
    n_rows, w_ext = u_ext.shape
    s_out_p = w_in.shape[0]
    h_dim = w_in.shape[1]
    w_half = h_dim // 2
    grid = (n_rows // BN,)
    has_skip = skip is not None

    def body(*refs):
        if has_skip:
            (u_r, sk_r, wc_r, g_r, b_r, win_r, bin_r, wm_r, h_r, hm_r) = refs
        else:
            (u_r, wc_r, g_r, b_r, win_r, bin_r, wm_r, h_r, hm_r) = refs
        t = _tree_conv(u_r[...], wc_r, s_out_p)
        mask = lax.broadcasted_iota(jnp.int32, t.shape, 1) < s_real
        tm = jnp.where(mask, t, 0.0)
        mu = jnp.sum(tm, axis=1, keepdims=True) / s_real
        dv = jnp.where(mask, t - mu, 0.0)
        var = jnp.sum(dv * dv, axis=1, keepdims=True) / s_real
        tl = (t - mu) / jnp.sqrt(var + 1e-5) * g_r[...] + b_r[...]
        a = jnp.where(tl > 0, tl, _expm1(jnp.where(tl > 0, 0.0, tl)))
        if has_skip:
            a = a + sk_r[...]
        h = jnp.maximum(_dot(a, win_r[...]) + bin_r[...], 0.0)
        h_r[...] = h
        hm = _dot(h, wm_r[...])
        hm_r[0] = hm[:, :w_half]
        hm_r[1] = hm[:, w_half:]

    in_specs = [pl.BlockSpec((BN, w_ext), lambda r: (r, 0))]
    args = [u_ext]
    if has_skip:
        in_specs.append(pl.BlockSpec((BN, s_out_p), lambda r: (r, 0)))
        args.append(skip)
    in_specs += [
        pl.BlockSpec((1, KW), lambda r: (0, 0)),
        pl.BlockSpec((1, s_out_p), lambda r: (0, 0)),
        pl.BlockSpec((1, s_out_p), lambda r: (0, 0)),
        pl.BlockSpec((s_out_p, h_dim), lambda r: (0, 0)),
        pl.BlockSpec((1, h_dim), lambda r: (0, 0)),
        pl.BlockSpec((h_dim, h_dim), lambda r: (0, 0)),
    ]
    args += [wc, ln_g, ln_b, w_in, b_in, w_msg]
    return pl.pallas_call(
        body,
        grid=grid,
        in_specs=in_specs,
        out_specs=[
            pl.BlockSpec((BN, h_dim), lambda r: (r, 0)),
            pl.BlockSpec((2, BN, w_half), lambda r: (0, r, 0)),
        ],
        out_shape=[
            jax.ShapeDtypeStruct((n_rows, h_dim), jnp.float32),
            jax.ShapeDtypeStruct((2, n_rows, w_half), jnp.float32),
        ],
        interpret=interpret,
    )(*args)


def _step_call(h, m2, wx, uh, b3, w_last, b_last, *, last, interpret=False):
    n_rows, h_dim = h.shape
    w_half = h_dim // 2
    grid = (n_rows // BN,)

    def body(*refs):
        if last:
            h_r, m_r, wx_r, uh_r, b3_r, wl_r, bl_r, o_r = refs
        else:
            h_r, m_r, wx_r, uh_r, b3_r, wl_r, hn_r, hm_r = refs
        h_v = h_r[...]
        m = jnp.concatenate([m_r[0], m_r[1]], axis=1)
        gx = _dot(m, wx_r[...]) + b3_r[...]
        gh = _dot(h_v, uh_r[...])
        z = jax.nn.sigmoid(gx[:, :h_dim] + gh[:, :h_dim])
        r = jax.nn.sigmoid(gx[:, h_dim:2 * h_dim] + gh[:, h_dim:2 * h_dim])
        cand = jnp.tanh(gx[:, 2 * h_dim:] + r * gh[:, 2 * h_dim:])
        hn = (1.0 - z) * cand + z * h_v
        if last:
            o_r[...] = _dot(hn, wl_r[...]) + bl_r[...]
        else:
            hn_r[...] = hn
            hm = _dot(hn, wl_r[...])
            hm_r[0] = hm[:, :w_half]
            hm_r[1] = hm[:, w_half:]

    g3 = 3 * h_dim
    in_specs = [
        pl.BlockSpec((BN, h_dim), lambda r: (r, 0)),
        pl.BlockSpec((2, BN, w_half), lambda r: (0, r, 0)),
        pl.BlockSpec((h_dim, g3), lambda r: (0, 0)),
        pl.BlockSpec((h_dim, g3), lambda r: (0, 0)),
        pl.BlockSpec((1, g3), lambda r: (0, 0)),
        pl.BlockSpec(w_last.shape, lambda r: (0, 0)),
    ]
    args = [h, m2, wx, uh, b3, w_last]
    if last:
        s_next = w_last.shape[1]
        in_specs.append(pl.BlockSpec((1, s_next), lambda r: (0, 0)))
        args.append(b_last)
        out_specs = pl.BlockSpec((BN, s_next), lambda r: (r, 0))
        out_shape = jax.ShapeDtypeStruct((n_rows, s_next), jnp.float32)
    else:
        out_specs = [
            pl.BlockSpec((BN, h_dim), lambda r: (r, 0)),
            pl.BlockSpec((2, BN, w_half), lambda r: (0, r, 0)),
        ]
        out_shape = [
            jax.ShapeDtypeStruct((n_rows, h_dim), jnp.float32),
            jax.ShapeDtypeStruct((2, n_rows, w_half), jnp.float32),
        ]
    return pl.pallas_call(
        body,
        grid=grid,
        in_specs=in_specs,
        out_specs=out_specs,
        out_shape=out_shape,
        interpret=interpret,
    )(*args)


@functools.lru_cache(maxsize=None)
def _make_seg(w_half, n_nodes, n_edges):
    e_sub = n_edges // NSUB
    full = e_sub // SC_CHUNK
    tail = e_sub % SC_CHUNK
    zrows = 16
    n_sub = (n_nodes // NSUB) // 8 * 8
    n_rem = n_nodes - NSUB * n_sub
    assert w_half == 128 and n_sub % zrows == 0 and n_rem % zrows == 0
    assert e_sub % 8 == 0 and tail % 16 == 0

    mesh = plsc.VectorSubcoreMesh(core_axis_name="c", subcore_axis_name="s")
    scratch = [
        pltpu.VMEM((SC_CHUNK,), jnp.int32),
        pltpu.VMEM((SC_CHUNK,), jnp.int32),
        pltpu.VMEM((SC_CHUNK, w_half), jnp.float32),
        pltpu.VMEM((zrows, w_half), jnp.float32),
        pltpu.VMEM_SHARED((n_nodes, w_half), jnp.float32),
        pltpu.SemaphoreType.DMA,
    ]
    if tail:
        scratch += [
            pltpu.VMEM((tail,), jnp.int32),
            pltpu.VMEM((tail,), jnp.int32),
            pltpu.VMEM((tail, w_half), jnp.float32),
        ]

    @functools.partial(
        pl.kernel,
        out_type=jax.ShapeDtypeStruct((NCORE * n_nodes, w_half), jnp.float32),
        mesh=mesh,
        scratch_types=scratch,
    )
    def seg(hm, src, dst, out, sidx, didx, rows, zbuf, acc, sem, *tails):
        c = lax.axis_index("c")
        s = lax.axis_index("s")
        coff = c * n_nodes
        for i in range(zrows):
            for j in range(w_half // 16):
                zbuf[i, pl.ds(j * 16, 16)] = jnp.zeros((16,), jnp.float32)

        def zero_body(k, carry):
            pltpu.sync_copy(zbuf, acc.at[pl.ds(s * n_sub + k * zrows, zrows)])
            return carry

        lax.fori_loop(0, n_sub // zrows, zero_body, 0)
        if n_rem:
            @pl.when(s == NSUB - 1)
            def _zero_rem():
                def zr(k, carry):
                    pltpu.sync_copy(
                        zbuf,
                        acc.at[pl.ds(NSUB * n_sub + k * zrows, zrows)])
                    return carry
                lax.fori_loop(0, n_rem // zrows, zr, 0)
        plsc.subcore_barrier()

        ebase = s * e_sub

        def edge_body(g, carry):
            off = ebase + g * SC_CHUNK
            pltpu.sync_copy(src.at[pl.ds(off, SC_CHUNK)], sidx)
            pltpu.sync_copy(dst.at[pl.ds(off, SC_CHUNK)], didx)
            for j in range(SC_CHUNK // 16):
                sidx[pl.ds(j * 16, 16)] = sidx[pl.ds(j * 16, 16)] + coff
            pltpu.async_copy(hm.at[sidx], rows, sem).wait()
            pltpu.sync_copy(rows, acc.at[didx], add=True)
            return carry

        lax.fori_loop(0, full, edge_body, 0)

        if tail:
            sidx_t, didx_t, rows_t = tails
            off = ebase + full * SC_CHUNK
            pltpu.sync_copy(src.at[pl.ds(off, tail)], sidx_t)
            pltpu.sync_copy(dst.at[pl.ds(off, tail)], didx_t)
            for j in range(tail // 16):
                sidx_t[pl.ds(j * 16, 16)] = sidx_t[pl.ds(j * 16, 16)] + coff
            pltpu.async_copy(hm.at[sidx_t], rows_t, sem).wait()
            pltpu.sync_copy(rows_t, acc.at[didx_t], add=True)

        plsc.subcore_barrier()
        pltpu.sync_copy(acc.at[pl.ds(s * n_sub, n_sub)],
                        out.at[pl.ds(coff + s * n_sub, n_sub)])
        if n_rem:
            @pl.when(s == NSUB - 1)
            def _write_rem():
                pltpu.sync_copy(acc.at[pl.ds(NSUB * n_sub, n_rem)],
                                out.at[pl.ds(coff + NSUB * n_sub, n_rem)])

    return seg


def _sc_seg(hm2, src, dst):
    two_n, w_half = hm2.shape
    seg = _make_seg(w_half, two_n // NCORE, src.shape[0])
    return seg(hm2, src, dst)


def _layer(u, skip, pk, s_in_real, s_out_real, transpose, src, dst,
           seg_fn, interpret):
    gnn = pk["gnn"]
    hid = s_out_real // 2
    s_out_p = _pad_up(s_out_real, 128)
    h_dim = H_DIM
    w_half = h_dim // 2
    g3 = 3 * h_dim

    w_ext = s_out_p + 128
    off = (KW - 1) if transpose else 0
    u_ext = jnp.zeros((u.shape[0], w_ext), jnp.float32)
    u_ext = u_ext.at[:, off:off + s_in_real].set(u[:, :s_in_real])
    wc = pk["conv_w"][0, 0, 0][None]

    ln_g = _pad1(pk["ln_g"], s_out_p)[None]
    ln_b = _pad1(pk["ln_b"], s_out_p)[None]
    w_in = _pad2(gnn["W_in"], s_out_p, h_dim)
    b_in = _pad1(gnn["b_in"], h_dim)[None]
    w_msg = _pad2(gnn["W_msg"], h_dim, h_dim)
    wx_p = jnp.zeros((h_dim, g3), jnp.float32)
    uh_p = jnp.zeros((h_dim, g3), jnp.float32)
    b3 = jnp.zeros((g3,), jnp.float32)
    for gi in range(3):
        wx_p = wx_p.at[:hid, gi * h_dim:gi * h_dim + hid].set(
            gnn["Wx"][:, gi * hid:(gi + 1) * hid])
        uh_p = uh_p.at[:hid, gi * h_dim:gi * h_dim + hid].set(
            gnn["Uh"][:, gi * hid:(gi + 1) * hid])
        b3 = b3.at[gi * h_dim:gi * h_dim + hid].set(
            gnn["b"][gi * hid:(gi + 1) * hid])
    b3 = b3[None]
    w_out = _pad2(gnn["W_out"], h_dim, s_out_p)
    b_out = _pad1(gnn["b_out"], s_out_p)[None]

    h, hm = _pre_call(u_ext, skip, wc, ln_g, ln_b, w_in, b_in, w_msg,
                      s_real=s_out_real, interpret=interpret)
    n_rows = u.shape[0]
    for st in range(4):
        m2 = seg_fn(hm.reshape(2 * n_rows, w_half), src, dst)
        m2 = m2.reshape(2, n_rows, w_half)
        if st == 3:
            return _step_call(h, m2, wx_p, uh_p, b3, w_out, b_out,
                              last=True, interpret=interpret)
        h, hm = _step_call(h, m2, wx_p, uh_p, b3, w_msg, None,
                           last=False, interpret=interpret)


def _forward(x, src, dst, params, seg_fn=_sc_seg, interpret=False):
    n_batch, _, n_per, f0 = x.shape
    u = x.reshape(n_batch * n_per, f0)
    order = jnp.argsort(dst, stable=True)
    src = jnp.take(src, order)
    dst = jnp.take(dst, order)
    skips = [u]
    s_in = f0
    for k in range(5):
        o = _layer(u, None, params["down"][k], s_in, s_in - (KW - 1), False,
                   src, dst, seg_fn, interpret)
        s_in -= (KW - 1)
        if k < 4:
            skips.append(o)
        u = o
    for k in range(5):
        o = _layer(u, skips[4 - k], params["up"][k], s_in, s_in + (KW - 1),
                   True, src, dst, seg_fn, interpret)
        s_in += (KW - 1)
        u = o
    return u


def kernel(x, edge_index, batch, params):
    del batch
    src = edge_index[0].astype(jnp.int32)
    dst = edge_index[1].astype(jnp.int32)
    out = _forward(x, src, dst, params)
    n_batch, _, n_per, f0 = x.shape
    return out[:, :f0].reshape(n_batch, n_per, f0)
